# baseline (speedup 1.0000x reference)
import functools

import jax
import jax.numpy as jnp
from jax import lax
from jax.experimental import pallas as pl
from jax.experimental.pallas import tpu as pltpu
from jax.experimental.pallas import tpu_sc as plsc

NC = 2
NS = 16
L = 16
NW = NC * NS


def _f32(shape):
    return jax.ShapeDtypeStruct(shape, jnp.float32)


def kernel(x, edge_index, W_base, b_base, W1, b1, W2, b2):
    N, DI = x.shape
    DO = W_base.shape[1]
    DH = W1.shape[1]
    E = edge_index.shape[1]

    NP = (N // (NS * L) + 1) * (NS * L)
    SL = NP // NS
    EW = E // NW
    CH = 128
    NB = 4
    KP = -(-EW // CH)
    KP = -(-KP // NB) * NB
    EP = KP * CH
    RB = 400
    GRID = N // RB

    mesh = plsc.VectorSubcoreMesh(
        core_axis_name="c", subcore_axis_name="s",
        num_cores=NC, num_subcores=NS)
    sc_params = pltpu.CompilerParams(
        needs_layout_passes=False, use_tc_tiling_on_sc=False)

    src = edge_index[0]
    dst = edge_index[1]
    pad = EP - EW
    srcp = jnp.pad(src.reshape(NW, EW), ((0, 0), (0, pad))).reshape(
        NW, KP, CH)
    dstp = jnp.pad(dst.reshape(NW, EW), ((0, 0), (0, pad)),
                   constant_values=N).reshape(NW, KP, CH)
    ones8 = jnp.ones((N, 8), jnp.float32)
    z8 = jnp.zeros((SL, 8), jnp.float32)
    z64 = jnp.zeros((SL, DO), jnp.float32)

    def make_prop(D):
        @functools.partial(
            pl.kernel,
            out_type=_f32((NC, NP, D)),
            mesh=mesh,
            compiler_params=sc_params,
            scratch_types=[
                pltpu.VMEM((KP, CH), jnp.int32),
                pltpu.VMEM((KP, CH), jnp.int32),
                [pltpu.VMEM((CH, D), jnp.float32) for _ in range(NB)],
                [pltpu.SemaphoreType.DMA for _ in range(NB)],
                [pltpu.SemaphoreType.DMA for _ in range(NB)],
                pltpu.VMEM_SHARED((NP, D), jnp.float32),
            ],
        )
        def k_prop(tab_hbm, srcp_hbm, dstp_hbm, zr_hbm, out_hbm,
                   src_v, dst_v, bufs, gsems, ssems, sh_v):
            c = lax.axis_index("c")
            s = lax.axis_index("s")
            wid = c * NS + s
            pltpu.sync_copy(srcp_hbm.at[wid], src_v)
            pltpu.sync_copy(dstp_hbm.at[wid], dst_v)
            pltpu.sync_copy(zr_hbm, sh_v.at[pl.ds(s * SL, SL)])
            plsc.subcore_barrier()

            def gather(j, b):
                pltpu.async_copy(tab_hbm.at[src_v.at[j]], bufs[b], gsems[b])

            def gather_wait(j, b):
                pltpu.make_async_copy(
                    tab_hbm.at[src_v.at[j]], bufs[b], gsems[b]).wait()

            def scatter(j, b):
                pltpu.async_copy(bufs[b], sh_v.at[dst_v.at[j]], ssems[b],
                                 add=True)

            def scatter_wait(j, b):
                pltpu.make_async_copy(
                    bufs[b], sh_v.at[dst_v.at[j]], ssems[b]).wait()

            for b in range(NB - 1):
                gather(b, b)
            gather_wait(0, 0)
            scatter(0, 0)
            gather(NB - 1, NB - 1)
            for b in range(1, NB):
                gather_wait(b, b)
                scatter(b, b)
                scatter_wait(b - 1, b - 1)
                gather(b - 1 + NB, b - 1)

            def ob(g, carry):
                for b in range(NB):
                    j = g * NB + b
                    jd = j - 1
                    gather_wait(j, b)
                    scatter(j, b)
                    scatter_wait(jd, (NB + b - 1) % NB)
                    gather(jd + NB, (NB + b - 1) % NB)
                return carry
            lax.fori_loop(1, KP // NB - 1, ob, 0)
            for b in range(NB):
                j = KP - NB + b
                gather_wait(j, b)
                scatter(j, b)
                scatter_wait(j - 1, (NB + b - 1) % NB)
                if j - 1 + NB < KP:
                    gather(j - 1 + NB, (NB + b - 1) % NB)
            scatter_wait(KP - 1, NB - 1)

            plsc.subcore_barrier()
            pltpu.sync_copy(sh_v.at[pl.ds(s * SL, SL)],
                            out_hbm.at[c, pl.ds(s * SL, SL)])

        return k_prop

    prop_d = make_prop(DO)
    prop_8 = make_prop(8)

    def tcmm_body(x_ref, w_ref, h0_ref):
        h0_ref[...] = jnp.dot(x_ref[...], w_ref[...],
                              preferred_element_type=jnp.float32)

    tc_mm = pl.pallas_call(
        tcmm_body,
        grid=(GRID,),
        in_specs=[
            pl.BlockSpec((RB, DI), lambda i: (i, 0)),
            pl.BlockSpec((DI, DO), lambda i: (0, 0)),
        ],
        out_specs=pl.BlockSpec((RB, DO), lambda i: (i, 0)),
        out_shape=_f32((N, DO)),
    )

    def tc1_body(deg_ref, h0_ref, h0p_ref, dinv_ref):
        deg = deg_ref[0][:, 0:1] + deg_ref[1][:, 0:1] + 1.0
        dinv = lax.rsqrt(deg)
        h0p_ref[...] = h0_ref[...] * dinv
        dinv_ref[...] = dinv

    tc1 = pl.pallas_call(
        tc1_body,
        grid=(GRID,),
        in_specs=[
            pl.BlockSpec((NC, RB, 8), lambda i: (0, i, 0)),
            pl.BlockSpec((RB, DO), lambda i: (i, 0)),
        ],
        out_specs=[
            pl.BlockSpec((RB, DO), lambda i: (i, 0)),
            pl.BlockSpec((RB, 1), lambda i: (i, 0)),
        ],
        out_shape=[_f32((N, DO)), _f32((N, 1))],
    )

    def tc2_body(acc_ref, h0p_ref, dinv_ref, w1_ref, bb_ref,
                 logist_ref, h1p_ref):
        dinv = dinv_ref[...]
        pre = acc_ref[0] + acc_ref[1] + h0p_ref[...]
        logist = pre * dinv + bb_ref[...]
        logist_ref[...] = logist
        h1p_ref[...] = jnp.dot(logist, w1_ref[...],
                               preferred_element_type=jnp.float32) * dinv

    tc2 = pl.pallas_call(
        tc2_body,
        grid=(GRID,),
        in_specs=[
            pl.BlockSpec((NC, RB, DO), lambda i: (0, i, 0)),
            pl.BlockSpec((RB, DO), lambda i: (i, 0)),
            pl.BlockSpec((RB, 1), lambda i: (i, 0)),
            pl.BlockSpec((DO, DH), lambda i: (0, 0)),
            pl.BlockSpec((1, DO), lambda i: (0, 0)),
        ],
        out_specs=[
            pl.BlockSpec((RB, DO), lambda i: (i, 0)),
            pl.BlockSpec((RB, DH), lambda i: (i, 0)),
        ],
        out_shape=[_f32((N, DO)), _f32((N, DH))],
    )

    def tc3_body(acc_ref, h1p_ref, dinv_ref, w2_ref, b1_ref,
                 sp8_ref, sp_ref):
        dinv = dinv_ref[...]
        h = jnp.maximum(
            (acc_ref[0] + acc_ref[1] + h1p_ref[...]) * dinv + b1_ref[...],
            0.0)
        sp = jnp.dot(h, w2_ref[...], preferred_element_type=jnp.float32) \
            * dinv
        sp_ref[...] = sp
        sp8_ref[...] = jnp.concatenate(
            [sp, jnp.zeros((RB, 7), jnp.float32)], axis=1)

    tc3 = pl.pallas_call(
        tc3_body,
        grid=(GRID,),
        in_specs=[
            pl.BlockSpec((NC, RB, DH), lambda i: (0, i, 0)),
            pl.BlockSpec((RB, DH), lambda i: (i, 0)),
            pl.BlockSpec((RB, 1), lambda i: (i, 0)),
            pl.BlockSpec((DH, 1), lambda i: (0, 0)),
            pl.BlockSpec((1, DH), lambda i: (0, 0)),
        ],
        out_specs=[
            pl.BlockSpec((RB, 8), lambda i: (i, 0)),
            pl.BlockSpec((RB, 1), lambda i: (i, 0)),
        ],
        out_shape=[_f32((N, 8)), _f32((N, 1))],
    )

    def tc4_body(accs_ref, sp_ref, dinv_ref, b2_ref, logist_ref, out_ref):
        t = (accs_ref[0][:, 0:1] + accs_ref[1][:, 0:1] + sp_ref[...]) \
            * dinv_ref[...] + b2_ref[0, 0]
        t = jnp.log(jnp.exp(t) + 1.1)
        out_ref[...] = logist_ref[...] * t

    tc4 = pl.pallas_call(
        tc4_body,
        grid=(GRID,),
        in_specs=[
            pl.BlockSpec((NC, RB, 8), lambda i: (0, i, 0)),
            pl.BlockSpec((RB, 1), lambda i: (i, 0)),
            pl.BlockSpec((RB, 1), lambda i: (i, 0)),
            pl.BlockSpec((1, 1), lambda i: (0, 0)),
            pl.BlockSpec((RB, DO), lambda i: (i, 0)),
        ],
        out_specs=pl.BlockSpec((RB, DO), lambda i: (i, 0)),
        out_shape=_f32((N, DO)),
    )

    deg8 = prop_8(ones8, srcp, dstp, z8)
    h0 = tc_mm(x, W_base)
    h0p, dinv_col = tc1(deg8, h0)
    acc0 = prop_d(h0p, srcp, dstp, z64)
    logist, h1p = tc2(acc0, h0p, dinv_col, W1, b_base.reshape(1, DO))
    acc1 = prop_d(h1p, srcp, dstp, z64)
    sp8, sp_col = tc3(acc1, h1p, dinv_col, W2, b1.reshape(1, DH))
    accs8 = prop_8(sp8, srcp, dstp, z8)
    return tc4(accs8, sp_col, dinv_col, b2.reshape(1, 1), logist)


# speedup vs baseline: 1.0130x; 1.0130x over previous
import functools

import jax
import jax.numpy as jnp
from jax import lax
from jax.experimental import pallas as pl
from jax.experimental.pallas import tpu as pltpu
from jax.experimental.pallas import tpu_sc as plsc

NC = 2
NS = 16
L = 16
NW = NC * NS


def _f32(shape):
    return jax.ShapeDtypeStruct(shape, jnp.float32)


def kernel(x, edge_index, W_base, b_base, W1, b1, W2, b2):
    N, DI = x.shape
    DO = W_base.shape[1]
    DH = W1.shape[1]
    E = edge_index.shape[1]

    NP = (N // (NS * L) + 1) * (NS * L)
    SL = NP // NS
    EW = E // NW
    CH = 128
    NB = 4
    KP = -(-EW // CH)
    KP = -(-KP // NB) * NB
    EP = KP * CH
    RB = 400
    GRID = N // RB

    mesh = plsc.VectorSubcoreMesh(
        core_axis_name="c", subcore_axis_name="s",
        num_cores=NC, num_subcores=NS)
    sc_params = pltpu.CompilerParams(
        needs_layout_passes=False, use_tc_tiling_on_sc=False)

    src = edge_index[0]
    dst = edge_index[1]
    pad = EP - EW
    srcp = jnp.pad(src.reshape(NW, EW), ((0, 0), (0, pad))).reshape(
        NW, KP, CH)
    dstp = jnp.pad(dst.reshape(NW, EW), ((0, 0), (0, pad)),
                   constant_values=N).reshape(NW, KP, CH)
    ones8 = jnp.ones((N, 8), jnp.float32)
    z8 = jnp.zeros((SL, 8), jnp.float32)
    z64 = jnp.zeros((SL, DO), jnp.float32)

    def make_prop(D):
        @functools.partial(
            pl.kernel,
            out_type=_f32((NC, NP, D)),
            mesh=mesh,
            compiler_params=sc_params,
            scratch_types=[
                pltpu.VMEM((KP, CH), jnp.int32),
                pltpu.VMEM((KP, CH), jnp.int32),
                [pltpu.VMEM((CH, D), jnp.float32) for _ in range(NB)],
                [pltpu.SemaphoreType.DMA for _ in range(NB)],
                pltpu.VMEM_SHARED((NP, D), jnp.float32),
            ],
        )
        def k_prop(tab_hbm, srcp_hbm, dstp_hbm, zr_hbm, out_hbm,
                   src_v, dst_v, bufs, gsems, sh_v):
            c = lax.axis_index("c")
            s = lax.axis_index("s")
            wid = c * NS + s
            pltpu.sync_copy(srcp_hbm.at[wid], src_v)
            pltpu.sync_copy(dstp_hbm.at[wid], dst_v)
            pltpu.sync_copy(zr_hbm, sh_v.at[pl.ds(s * SL, SL)])
            plsc.subcore_barrier()

            for b in range(NB):
                pltpu.async_copy(tab_hbm.at[src_v.at[b]], bufs[b], gsems[b])

            def ob(g, carry):
                for b in range(NB):
                    j = g * NB + b
                    pltpu.make_async_copy(
                        tab_hbm.at[src_v.at[j]], bufs[b], gsems[b]).wait()
                    pltpu.sync_copy(bufs[b], sh_v.at[dst_v.at[j]], add=True)
                    pltpu.async_copy(
                        tab_hbm.at[src_v.at[j + NB]], bufs[b], gsems[b])
                return carry
            lax.fori_loop(0, KP // NB - 1, ob, 0)
            for b in range(NB):
                j = KP - NB + b
                pltpu.make_async_copy(
                    tab_hbm.at[src_v.at[j]], bufs[b], gsems[b]).wait()
                pltpu.sync_copy(bufs[b], sh_v.at[dst_v.at[j]], add=True)

            plsc.subcore_barrier()
            pltpu.sync_copy(sh_v.at[pl.ds(s * SL, SL)],
                            out_hbm.at[c, pl.ds(s * SL, SL)])

        return k_prop

    prop_d = make_prop(DO)
    prop_8 = make_prop(8)

    def tc1_body(deg_ref, x_ref, w_ref, h0p_ref, dinv_ref):
        deg = deg_ref[0][:, 0:1] + deg_ref[1][:, 0:1] + 1.0
        dinv = lax.rsqrt(deg)
        h0 = jnp.dot(x_ref[...], w_ref[...],
                     preferred_element_type=jnp.float32)
        h0p_ref[...] = h0 * dinv
        dinv_ref[...] = dinv

    tc1 = pl.pallas_call(
        tc1_body,
        grid=(GRID,),
        in_specs=[
            pl.BlockSpec((NC, RB, 8), lambda i: (0, i, 0)),
            pl.BlockSpec((RB, DI), lambda i: (i, 0)),
            pl.BlockSpec((DI, DO), lambda i: (0, 0)),
        ],
        out_specs=[
            pl.BlockSpec((RB, DO), lambda i: (i, 0)),
            pl.BlockSpec((RB, 1), lambda i: (i, 0)),
        ],
        out_shape=[_f32((N, DO)), _f32((N, 1))],
    )

    def tc2_body(acc_ref, h0p_ref, dinv_ref, w1_ref, bb_ref,
                 logist_ref, h1p_ref):
        dinv = dinv_ref[...]
        pre = acc_ref[0] + acc_ref[1] + h0p_ref[...]
        logist = pre * dinv + bb_ref[...]
        logist_ref[...] = logist
        h1p_ref[...] = jnp.dot(logist, w1_ref[...],
                               preferred_element_type=jnp.float32) * dinv

    tc2 = pl.pallas_call(
        tc2_body,
        grid=(GRID,),
        in_specs=[
            pl.BlockSpec((NC, RB, DO), lambda i: (0, i, 0)),
            pl.BlockSpec((RB, DO), lambda i: (i, 0)),
            pl.BlockSpec((RB, 1), lambda i: (i, 0)),
            pl.BlockSpec((DO, DH), lambda i: (0, 0)),
            pl.BlockSpec((1, DO), lambda i: (0, 0)),
        ],
        out_specs=[
            pl.BlockSpec((RB, DO), lambda i: (i, 0)),
            pl.BlockSpec((RB, DH), lambda i: (i, 0)),
        ],
        out_shape=[_f32((N, DO)), _f32((N, DH))],
    )

    def tc3_body(acc_ref, h1p_ref, dinv_ref, w2_ref, b1_ref,
                 sp8_ref, sp_ref):
        dinv = dinv_ref[...]
        h = jnp.maximum(
            (acc_ref[0] + acc_ref[1] + h1p_ref[...]) * dinv + b1_ref[...],
            0.0)
        sp = jnp.dot(h, w2_ref[...], preferred_element_type=jnp.float32) \
            * dinv
        sp_ref[...] = sp
        sp8_ref[...] = jnp.concatenate(
            [sp, jnp.zeros((RB, 7), jnp.float32)], axis=1)

    tc3 = pl.pallas_call(
        tc3_body,
        grid=(GRID,),
        in_specs=[
            pl.BlockSpec((NC, RB, DH), lambda i: (0, i, 0)),
            pl.BlockSpec((RB, DH), lambda i: (i, 0)),
            pl.BlockSpec((RB, 1), lambda i: (i, 0)),
            pl.BlockSpec((DH, 1), lambda i: (0, 0)),
            pl.BlockSpec((1, DH), lambda i: (0, 0)),
        ],
        out_specs=[
            pl.BlockSpec((RB, 8), lambda i: (i, 0)),
            pl.BlockSpec((RB, 1), lambda i: (i, 0)),
        ],
        out_shape=[_f32((N, 8)), _f32((N, 1))],
    )

    def tc4_body(accs_ref, sp_ref, dinv_ref, b2_ref, logist_ref, out_ref):
        t = (accs_ref[0][:, 0:1] + accs_ref[1][:, 0:1] + sp_ref[...]) \
            * dinv_ref[...] + b2_ref[0, 0]
        t = jnp.log(jnp.exp(t) + 1.1)
        out_ref[...] = logist_ref[...] * t

    tc4 = pl.pallas_call(
        tc4_body,
        grid=(GRID,),
        in_specs=[
            pl.BlockSpec((NC, RB, 8), lambda i: (0, i, 0)),
            pl.BlockSpec((RB, 1), lambda i: (i, 0)),
            pl.BlockSpec((RB, 1), lambda i: (i, 0)),
            pl.BlockSpec((1, 1), lambda i: (0, 0)),
            pl.BlockSpec((RB, DO), lambda i: (i, 0)),
        ],
        out_specs=pl.BlockSpec((RB, DO), lambda i: (i, 0)),
        out_shape=_f32((N, DO)),
    )

    deg8 = prop_8(ones8, srcp, dstp, z8)
    h0p, dinv_col = tc1(deg8, x, W_base)
    acc0 = prop_d(h0p, srcp, dstp, z64)
    logist, h1p = tc2(acc0, h0p, dinv_col, W1, b_base.reshape(1, DO))
    acc1 = prop_d(h1p, srcp, dstp, z64)
    sp8, sp_col = tc3(acc1, h1p, dinv_col, W2, b1.reshape(1, DH))
    accs8 = prop_8(sp8, srcp, dstp, z8)
    return tc4(accs8, sp_col, dinv_col, b2.reshape(1, 1), logist)


# speedup vs baseline: 1.1496x; 1.1348x over previous
import functools

import jax
import jax.numpy as jnp
from jax import lax
from jax.experimental import pallas as pl
from jax.experimental.pallas import tpu as pltpu
from jax.experimental.pallas import tpu_sc as plsc

NC = 2
NS = 16
L = 16
NW = NC * NS


def _f32(shape):
    return jax.ShapeDtypeStruct(shape, jnp.float32)


def kernel(x, edge_index, W_base, b_base, W1, b1, W2, b2):
    N, DI = x.shape
    DO = W_base.shape[1]
    DH = W1.shape[1]
    E = edge_index.shape[1]

    NP = (N // (NS * L) + 1) * (NS * L)
    SL = NP // NS
    EW = E // NW
    CH = 128
    NB = 4
    KP = -(-EW // CH)
    KP = -(-KP // NB) * NB
    EP = KP * CH
    RB = 1000
    GRID = N // RB

    mesh = plsc.VectorSubcoreMesh(
        core_axis_name="c", subcore_axis_name="s",
        num_cores=NC, num_subcores=NS)
    sc_params = pltpu.CompilerParams(
        needs_layout_passes=False, use_tc_tiling_on_sc=False)

    src = edge_index[0]
    dst = edge_index[1]
    pad = EP - EW
    srcp = jnp.pad(src.reshape(NW, EW), ((0, 0), (0, pad))).reshape(
        NW, KP, CH)
    dstp = jnp.pad(dst.reshape(NW, EW), ((0, 0), (0, pad)),
                   constant_values=N).reshape(NW, KP, CH)
    ones8 = jnp.ones((CH, 8), jnp.float32)
    z8 = jnp.zeros((SL, 8), jnp.float32)
    z64 = jnp.zeros((SL, DO), jnp.float32)

    def make_prop(D):
        @functools.partial(
            pl.kernel,
            out_type=_f32((NC, NP, D)),
            mesh=mesh,
            compiler_params=sc_params,
            scratch_types=[
                pltpu.VMEM((KP, CH), jnp.int32),
                pltpu.VMEM((KP, CH), jnp.int32),
                [pltpu.VMEM((CH, D), jnp.float32) for _ in range(NB)],
                [pltpu.SemaphoreType.DMA for _ in range(NB)],
                pltpu.VMEM_SHARED((NP, D), jnp.float32),
            ],
        )
        def k_prop(tab_hbm, srcp_hbm, dstp_hbm, zr_hbm, out_hbm,
                   src_v, dst_v, bufs, gsems, sh_v):
            c = lax.axis_index("c")
            s = lax.axis_index("s")
            wid = c * NS + s
            pltpu.sync_copy(srcp_hbm.at[wid], src_v)
            pltpu.sync_copy(dstp_hbm.at[wid], dst_v)
            pltpu.sync_copy(zr_hbm, sh_v.at[pl.ds(s * SL, SL)])
            plsc.subcore_barrier()

            for b in range(NB):
                pltpu.async_copy(tab_hbm.at[src_v.at[b]], bufs[b], gsems[b])

            def ob(g, carry):
                for b in range(NB):
                    j = g * NB + b
                    pltpu.make_async_copy(
                        tab_hbm.at[src_v.at[j]], bufs[b], gsems[b]).wait()
                    pltpu.sync_copy(bufs[b], sh_v.at[dst_v.at[j]], add=True)
                    pltpu.async_copy(
                        tab_hbm.at[src_v.at[j + NB]], bufs[b], gsems[b])
                return carry
            lax.fori_loop(0, KP // NB - 1, ob, 0)
            for b in range(NB):
                j = KP - NB + b
                pltpu.make_async_copy(
                    tab_hbm.at[src_v.at[j]], bufs[b], gsems[b]).wait()
                pltpu.sync_copy(bufs[b], sh_v.at[dst_v.at[j]], add=True)

            plsc.subcore_barrier()
            pltpu.sync_copy(sh_v.at[pl.ds(s * SL, SL)],
                            out_hbm.at[c, pl.ds(s * SL, SL)])

        return k_prop

    prop_d = make_prop(DO)
    prop_8 = make_prop(8)

    @functools.partial(
        pl.kernel,
        out_type=_f32((NC, NP, 8)),
        mesh=mesh,
        compiler_params=sc_params,
        scratch_types=[
            pltpu.VMEM((KP, CH), jnp.int32),
            pltpu.VMEM((CH, 8), jnp.float32),
            [pltpu.SemaphoreType.DMA for _ in range(NB)],
            pltpu.VMEM_SHARED((NP, 8), jnp.float32),
        ],
    )
    def k_deg(ones_hbm, dstp_hbm, zr_hbm, out_hbm,
              dst_v, ones_v, sems, sh_v):
        c = lax.axis_index("c")
        s = lax.axis_index("s")
        wid = c * NS + s
        pltpu.sync_copy(dstp_hbm.at[wid], dst_v)
        pltpu.sync_copy(ones_hbm, ones_v)
        pltpu.sync_copy(zr_hbm, sh_v.at[pl.ds(s * SL, SL)])
        plsc.subcore_barrier()

        def scat(j, b):
            pltpu.async_copy(ones_v, sh_v.at[dst_v.at[j]], sems[b],
                             add=True)

        def scat_wait(j, b):
            pltpu.make_async_copy(
                ones_v, sh_v.at[dst_v.at[j]], sems[b]).wait()

        for b in range(NB):
            scat(b, b)

        def ob(g, carry):
            for b in range(NB):
                j = g * NB + b
                scat_wait(j, b)
                scat(j + NB, b)
            return carry
        lax.fori_loop(0, KP // NB - 1, ob, 0)
        for b in range(NB):
            scat_wait(KP - NB + b, b)

        plsc.subcore_barrier()
        pltpu.sync_copy(sh_v.at[pl.ds(s * SL, SL)],
                        out_hbm.at[c, pl.ds(s * SL, SL)])

    def tc1_body(deg_ref, x_ref, w_ref, h0p_ref, dinv_ref):
        deg = deg_ref[0][:, 0:1] + deg_ref[1][:, 0:1] + 1.0
        dinv = lax.rsqrt(deg)
        h0 = jnp.dot(x_ref[...], w_ref[...],
                     preferred_element_type=jnp.float32)
        h0p_ref[...] = h0 * dinv
        dinv_ref[...] = dinv

    tc1 = pl.pallas_call(
        tc1_body,
        grid=(GRID,),
        in_specs=[
            pl.BlockSpec((NC, RB, 8), lambda i: (0, i, 0)),
            pl.BlockSpec((RB, DI), lambda i: (i, 0)),
            pl.BlockSpec((DI, DO), lambda i: (0, 0)),
        ],
        out_specs=[
            pl.BlockSpec((RB, DO), lambda i: (i, 0)),
            pl.BlockSpec((RB, 1), lambda i: (i, 0)),
        ],
        out_shape=[_f32((N, DO)), _f32((N, 1))],
    )

    def tc2_body(acc_ref, h0p_ref, dinv_ref, w1_ref, bb_ref,
                 logist_ref, h1p_ref):
        dinv = dinv_ref[...]
        pre = acc_ref[0] + acc_ref[1] + h0p_ref[...]
        logist = pre * dinv + bb_ref[...]
        logist_ref[...] = logist
        h1p_ref[...] = jnp.dot(logist, w1_ref[...],
                               preferred_element_type=jnp.float32) * dinv

    tc2 = pl.pallas_call(
        tc2_body,
        grid=(GRID,),
        in_specs=[
            pl.BlockSpec((NC, RB, DO), lambda i: (0, i, 0)),
            pl.BlockSpec((RB, DO), lambda i: (i, 0)),
            pl.BlockSpec((RB, 1), lambda i: (i, 0)),
            pl.BlockSpec((DO, DH), lambda i: (0, 0)),
            pl.BlockSpec((1, DO), lambda i: (0, 0)),
        ],
        out_specs=[
            pl.BlockSpec((RB, DO), lambda i: (i, 0)),
            pl.BlockSpec((RB, DH), lambda i: (i, 0)),
        ],
        out_shape=[_f32((N, DO)), _f32((N, DH))],
    )

    def tc3_body(acc_ref, h1p_ref, dinv_ref, w2_ref, b1_ref,
                 sp8_ref, sp_ref):
        dinv = dinv_ref[...]
        h = jnp.maximum(
            (acc_ref[0] + acc_ref[1] + h1p_ref[...]) * dinv + b1_ref[...],
            0.0)
        sp = jnp.dot(h, w2_ref[...], preferred_element_type=jnp.float32) \
            * dinv
        sp_ref[...] = sp
        sp8_ref[...] = jnp.concatenate(
            [sp, jnp.zeros((RB, 7), jnp.float32)], axis=1)

    tc3 = pl.pallas_call(
        tc3_body,
        grid=(GRID,),
        in_specs=[
            pl.BlockSpec((NC, RB, DH), lambda i: (0, i, 0)),
            pl.BlockSpec((RB, DH), lambda i: (i, 0)),
            pl.BlockSpec((RB, 1), lambda i: (i, 0)),
            pl.BlockSpec((DH, 1), lambda i: (0, 0)),
            pl.BlockSpec((1, DH), lambda i: (0, 0)),
        ],
        out_specs=[
            pl.BlockSpec((RB, 8), lambda i: (i, 0)),
            pl.BlockSpec((RB, 1), lambda i: (i, 0)),
        ],
        out_shape=[_f32((N, 8)), _f32((N, 1))],
    )

    def tc4_body(accs_ref, sp_ref, dinv_ref, b2_ref, logist_ref, out_ref):
        t = (accs_ref[0][:, 0:1] + accs_ref[1][:, 0:1] + sp_ref[...]) \
            * dinv_ref[...] + b2_ref[0, 0]
        t = jnp.log(jnp.exp(t) + 1.1)
        out_ref[...] = logist_ref[...] * t

    tc4 = pl.pallas_call(
        tc4_body,
        grid=(GRID,),
        in_specs=[
            pl.BlockSpec((NC, RB, 8), lambda i: (0, i, 0)),
            pl.BlockSpec((RB, 1), lambda i: (i, 0)),
            pl.BlockSpec((RB, 1), lambda i: (i, 0)),
            pl.BlockSpec((1, 1), lambda i: (0, 0)),
            pl.BlockSpec((RB, DO), lambda i: (i, 0)),
        ],
        out_specs=pl.BlockSpec((RB, DO), lambda i: (i, 0)),
        out_shape=_f32((N, DO)),
    )

    deg8 = k_deg(ones8, dstp, z8)
    h0p, dinv_col = tc1(deg8, x, W_base)
    acc0 = prop_d(h0p, srcp, dstp, z64)
    logist, h1p = tc2(acc0, h0p, dinv_col, W1, b_base.reshape(1, DO))
    acc1 = prop_d(h1p, srcp, dstp, z64)
    sp8, sp_col = tc3(acc1, h1p, dinv_col, W2, b1.reshape(1, DH))
    accs8 = prop_8(sp8, srcp, dstp, z8)
    return tc4(accs8, sp_col, dinv_col, b2.reshape(1, 1), logist)


# speedup vs baseline: 1.1647x; 1.0131x over previous
import functools

import jax
import jax.numpy as jnp
from jax import lax
from jax.experimental import pallas as pl
from jax.experimental.pallas import tpu as pltpu
from jax.experimental.pallas import tpu_sc as plsc

NC = 2
NS = 16
L = 16
NW = NC * NS


def _f32(shape):
    return jax.ShapeDtypeStruct(shape, jnp.float32)


def kernel(x, edge_index, W_base, b_base, W1, b1, W2, b2):
    N, DI = x.shape
    DO = W_base.shape[1]
    DH = W1.shape[1]
    E = edge_index.shape[1]

    NP = (N // (NS * L) + 1) * (NS * L)
    SL = NP // NS
    EW = E // NW
    CH = 128
    NB = 4
    KP = -(-EW // CH)
    KP = -(-KP // NB) * NB
    EP = KP * CH
    RB = 2000
    GRID = N // RB

    mesh = plsc.VectorSubcoreMesh(
        core_axis_name="c", subcore_axis_name="s",
        num_cores=NC, num_subcores=NS)
    sc_params = pltpu.CompilerParams(
        needs_layout_passes=False, use_tc_tiling_on_sc=False)

    src = edge_index[0]
    dst = edge_index[1]
    pad = EP - EW
    srcp = jnp.pad(src.reshape(NW, EW), ((0, 0), (0, pad))).reshape(
        NW, KP, CH)
    dstp = jnp.pad(dst.reshape(NW, EW), ((0, 0), (0, pad)),
                   constant_values=N).reshape(NW, KP, CH)
    ones8 = jnp.ones((CH, 8), jnp.float32)
    z8 = jnp.zeros((SL, 8), jnp.float32)
    z64 = jnp.zeros((SL, DO), jnp.float32)

    def make_prop(D):
        @functools.partial(
            pl.kernel,
            out_type=_f32((NC, NP, D)),
            mesh=mesh,
            compiler_params=sc_params,
            scratch_types=[
                pltpu.VMEM((KP, CH), jnp.int32),
                pltpu.VMEM((KP, CH), jnp.int32),
                [pltpu.VMEM((CH, D), jnp.float32) for _ in range(NB)],
                [pltpu.SemaphoreType.DMA for _ in range(NB)],
                pltpu.VMEM_SHARED((NP, D), jnp.float32),
            ],
        )
        def k_prop(tab_hbm, srcp_hbm, dstp_hbm, zr_hbm, out_hbm,
                   src_v, dst_v, bufs, gsems, sh_v):
            c = lax.axis_index("c")
            s = lax.axis_index("s")
            wid = c * NS + s
            pltpu.sync_copy(srcp_hbm.at[wid], src_v)
            pltpu.sync_copy(dstp_hbm.at[wid], dst_v)
            pltpu.sync_copy(zr_hbm, sh_v.at[pl.ds(s * SL, SL)])
            plsc.subcore_barrier()

            for b in range(NB):
                pltpu.async_copy(tab_hbm.at[src_v.at[b]], bufs[b], gsems[b])

            def ob(g, carry):
                for b in range(NB):
                    j = g * NB + b
                    pltpu.make_async_copy(
                        tab_hbm.at[src_v.at[j]], bufs[b], gsems[b]).wait()
                    pltpu.sync_copy(bufs[b], sh_v.at[dst_v.at[j]], add=True)
                    pltpu.async_copy(
                        tab_hbm.at[src_v.at[j + NB]], bufs[b], gsems[b])
                return carry
            lax.fori_loop(0, KP // NB - 1, ob, 0)
            for b in range(NB):
                j = KP - NB + b
                pltpu.make_async_copy(
                    tab_hbm.at[src_v.at[j]], bufs[b], gsems[b]).wait()
                pltpu.sync_copy(bufs[b], sh_v.at[dst_v.at[j]], add=True)

            plsc.subcore_barrier()
            pltpu.sync_copy(sh_v.at[pl.ds(s * SL, SL)],
                            out_hbm.at[c, pl.ds(s * SL, SL)])

        return k_prop

    prop_d = make_prop(DO)
    prop_8 = make_prop(8)

    @functools.partial(
        pl.kernel,
        out_type=_f32((NC, NP, 8)),
        mesh=mesh,
        compiler_params=sc_params,
        scratch_types=[
            pltpu.VMEM((KP, CH), jnp.int32),
            pltpu.VMEM((CH, 8), jnp.float32),
            [pltpu.SemaphoreType.DMA for _ in range(NB)],
            pltpu.VMEM_SHARED((NP, 8), jnp.float32),
        ],
    )
    def k_deg(ones_hbm, dstp_hbm, zr_hbm, out_hbm,
              dst_v, ones_v, sems, sh_v):
        c = lax.axis_index("c")
        s = lax.axis_index("s")
        wid = c * NS + s
        pltpu.sync_copy(dstp_hbm.at[wid], dst_v)
        pltpu.sync_copy(ones_hbm, ones_v)
        pltpu.sync_copy(zr_hbm, sh_v.at[pl.ds(s * SL, SL)])
        plsc.subcore_barrier()

        def scat(j, b):
            pltpu.async_copy(ones_v, sh_v.at[dst_v.at[j]], sems[b],
                             add=True)

        def scat_wait(j, b):
            pltpu.make_async_copy(
                ones_v, sh_v.at[dst_v.at[j]], sems[b]).wait()

        for b in range(NB):
            scat(b, b)

        def ob(g, carry):
            for b in range(NB):
                j = g * NB + b
                scat_wait(j, b)
                scat(j + NB, b)
            return carry
        lax.fori_loop(0, KP // NB - 1, ob, 0)
        for b in range(NB):
            scat_wait(KP - NB + b, b)

        plsc.subcore_barrier()
        pltpu.sync_copy(sh_v.at[pl.ds(s * SL, SL)],
                        out_hbm.at[c, pl.ds(s * SL, SL)])

    def tc1_body(deg_ref, x_ref, w_ref, h0p_ref, dinv_ref):
        deg = deg_ref[0][:, 0:1] + deg_ref[1][:, 0:1] + 1.0
        dinv = lax.rsqrt(deg)
        h0 = jnp.dot(x_ref[...], w_ref[...],
                     preferred_element_type=jnp.float32)
        h0p_ref[...] = h0 * dinv
        dinv_ref[...] = dinv

    tc1 = pl.pallas_call(
        tc1_body,
        grid=(GRID,),
        in_specs=[
            pl.BlockSpec((NC, RB, 8), lambda i: (0, i, 0)),
            pl.BlockSpec((RB, DI), lambda i: (i, 0)),
            pl.BlockSpec((DI, DO), lambda i: (0, 0)),
        ],
        out_specs=[
            pl.BlockSpec((RB, DO), lambda i: (i, 0)),
            pl.BlockSpec((RB, 1), lambda i: (i, 0)),
        ],
        out_shape=[_f32((N, DO)), _f32((N, 1))],
    )

    def tc2_body(acc_ref, h0p_ref, dinv_ref, w1_ref, bb_ref,
                 logist_ref, h1p_ref):
        dinv = dinv_ref[...]
        pre = acc_ref[0] + acc_ref[1] + h0p_ref[...]
        logist = pre * dinv + bb_ref[...]
        logist_ref[...] = logist
        h1p_ref[...] = jnp.dot(logist, w1_ref[...],
                               preferred_element_type=jnp.float32) * dinv

    tc2 = pl.pallas_call(
        tc2_body,
        grid=(GRID,),
        in_specs=[
            pl.BlockSpec((NC, RB, DO), lambda i: (0, i, 0)),
            pl.BlockSpec((RB, DO), lambda i: (i, 0)),
            pl.BlockSpec((RB, 1), lambda i: (i, 0)),
            pl.BlockSpec((DO, DH), lambda i: (0, 0)),
            pl.BlockSpec((1, DO), lambda i: (0, 0)),
        ],
        out_specs=[
            pl.BlockSpec((RB, DO), lambda i: (i, 0)),
            pl.BlockSpec((RB, DH), lambda i: (i, 0)),
        ],
        out_shape=[_f32((N, DO)), _f32((N, DH))],
    )

    def tc3_body(acc_ref, h1p_ref, dinv_ref, w2_ref, b1_ref,
                 sp8_ref, sp_ref):
        dinv = dinv_ref[...]
        h = jnp.maximum(
            (acc_ref[0] + acc_ref[1] + h1p_ref[...]) * dinv + b1_ref[...],
            0.0)
        sp = jnp.dot(h, w2_ref[...], preferred_element_type=jnp.float32) \
            * dinv
        sp_ref[...] = sp
        sp8_ref[...] = jnp.concatenate(
            [sp, jnp.zeros((RB, 7), jnp.float32)], axis=1)

    tc3 = pl.pallas_call(
        tc3_body,
        grid=(GRID,),
        in_specs=[
            pl.BlockSpec((NC, RB, DH), lambda i: (0, i, 0)),
            pl.BlockSpec((RB, DH), lambda i: (i, 0)),
            pl.BlockSpec((RB, 1), lambda i: (i, 0)),
            pl.BlockSpec((DH, 1), lambda i: (0, 0)),
            pl.BlockSpec((1, DH), lambda i: (0, 0)),
        ],
        out_specs=[
            pl.BlockSpec((RB, 8), lambda i: (i, 0)),
            pl.BlockSpec((RB, 1), lambda i: (i, 0)),
        ],
        out_shape=[_f32((N, 8)), _f32((N, 1))],
    )

    def tc4_body(accs_ref, sp_ref, dinv_ref, b2_ref, logist_ref, out_ref):
        t = (accs_ref[0][:, 0:1] + accs_ref[1][:, 0:1] + sp_ref[...]) \
            * dinv_ref[...] + b2_ref[0, 0]
        t = jnp.log(jnp.exp(t) + 1.1)
        out_ref[...] = logist_ref[...] * t

    tc4 = pl.pallas_call(
        tc4_body,
        grid=(GRID,),
        in_specs=[
            pl.BlockSpec((NC, RB, 8), lambda i: (0, i, 0)),
            pl.BlockSpec((RB, 1), lambda i: (i, 0)),
            pl.BlockSpec((RB, 1), lambda i: (i, 0)),
            pl.BlockSpec((1, 1), lambda i: (0, 0)),
            pl.BlockSpec((RB, DO), lambda i: (i, 0)),
        ],
        out_specs=pl.BlockSpec((RB, DO), lambda i: (i, 0)),
        out_shape=_f32((N, DO)),
    )

    deg8 = k_deg(ones8, dstp, z8)
    h0p, dinv_col = tc1(deg8, x, W_base)
    acc0 = prop_d(h0p, srcp, dstp, z64)
    logist, h1p = tc2(acc0, h0p, dinv_col, W1, b_base.reshape(1, DO))
    acc1 = prop_d(h1p, srcp, dstp, z64)
    sp8, sp_col = tc3(acc1, h1p, dinv_col, W2, b1.reshape(1, DH))
    accs8 = prop_8(sp8, srcp, dstp, z8)
    return tc4(accs8, sp_col, dinv_col, b2.reshape(1, 1), logist)
